# Initial kernel scaffold; baseline (speedup 1.0000x reference)
import jax
import jax.numpy as jnp
from jax.experimental import pallas as pl


def kernel(x, edge_index, graph_ids, W0, b0, W1, b1, W2, b2, W3, b3, Wg, bg):
    raise NotImplementedError("write your pallas kernel here")


# speedup vs baseline: 4.9565x; 4.9565x over previous
import jax
import jax.numpy as jnp
from jax import lax
from jax.experimental import pallas as pl
from jax.experimental.pallas import tpu as pltpu
from jax.experimental.pallas import tpu_sc as plsc

NN = 50000
NR = 51200
NC = 2
NS = 16
CH = 128
KF = 16
EE = 800000
EP = 819200
SSP = EP // (NS * CH * KF)
TCH = EP // CH
RPT = NR // NS
DCH = RPT // CH
GG = 500
DG = 16
F32 = jnp.float32

_SC_PARAMS = pltpu.CompilerParams(use_tc_tiling_on_sc=False)


def _sc_mesh():
    return plsc.VectorSubcoreMesh(core_axis_name="c", subcore_axis_name="s")


def _zero_rows(buf):
    def zrow(r, _):
        buf[r, pl.ds(0, 16)] = jnp.zeros((16,), F32)
        return 0
    lax.fori_loop(0, CH, zrow, 0)


def _make_prop(ngrp, scaled_out):
    ph = ngrp // NC
    n_out = 2 if scaled_out else 1
    out_type = [jax.ShapeDtypeStruct((ngrp, NR, DG), F32)
                for _ in range(n_out)]
    scratch = [
        pltpu.VMEM((KF, CH), jnp.int32),
        pltpu.VMEM((KF, CH), jnp.int32),
        pltpu.VMEM((KF, CH, DG), F32),
        pltpu.VMEM((CH, DG), F32),
        pltpu.VMEM((CH, DG), F32),
        pltpu.VMEM((CH,), F32),
        pltpu.VMEM_SHARED((NR, DG), F32),
        pltpu.SemaphoreType.DMA,
        pltpu.SemaphoreType.DMA,
    ]

    def body(*refs):
        if scaled_out:
            table, srcg, dst2, scale_h = refs[:4]
            outs = refs[4:4 + n_out]
            sidx, didx, gbuf, rbuf, obuf, scv, acc, gsem, ssem = refs[4 + n_out:]
        else:
            table, srcg, dst2 = refs[:3]
            outs = refs[3:3 + n_out]
            sidx, didx, gbuf, rbuf, obuf, scv, acc, gsem, ssem = refs[3 + n_out:]
        out_raw = outs[0]
        c = lax.axis_index("c")
        s = lax.axis_index("s")
        tile_r0 = s * RPT
        chunk0 = s * (SSP * KF)
        _zero_rows(rbuf)

        for p in range(ph):
            g = c * ph + p

            def zchunk(k, _):
                pltpu.sync_copy(rbuf, acc.at[pl.ds(tile_r0 + k * CH, CH)])
                return 0
            lax.fori_loop(0, DCH, zchunk, 0)
            plsc.subcore_barrier()

            def sstep(ss, _):
                row0 = chunk0 + ss * KF
                pltpu.sync_copy(srcg.at[g, pl.ds(row0, KF)], sidx)
                pltpu.sync_copy(dst2.at[pl.ds(row0, KF)], didx)
                gds = [pltpu.async_copy(table.at[sidx.at[j]], gbuf.at[j],
                                        gsem)
                       for j in range(KF)]
                for dd in gds:
                    dd.wait()
                sds = [pltpu.async_copy(gbuf.at[j], acc.at[didx.at[j]], ssem,
                                        add=True)
                       for j in range(KF)]
                for dd in sds:
                    dd.wait()
                return 0
            lax.fori_loop(0, SSP, sstep, 0)
            plsc.subcore_barrier()

            def dchunk(k, _):
                r0 = tile_r0 + k * CH
                pltpu.sync_copy(acc.at[pl.ds(r0, CH)], rbuf)
                pltpu.sync_copy(rbuf, out_raw.at[g, pl.ds(r0, CH)])
                if scaled_out:
                    pltpu.sync_copy(scale_h.at[pl.ds(r0, CH)], scv)

                    def sgrp(rg, _):
                        rr = rg * 16
                        sv16 = scv[pl.ds(rr, 16)]
                        for kk in range(16):
                            obuf[rr + kk, pl.ds(0, 16)] = (
                                rbuf[rr + kk, pl.ds(0, 16)] * sv16[kk])
                        return 0
                    lax.fori_loop(0, CH // 16, sgrp, 0)
                    pltpu.sync_copy(obuf, outs[1].at[g, pl.ds(r0, CH)])
                return 0
            lax.fori_loop(0, DCH, dchunk, 0)
            if p + 1 < ph:
                _zero_rows(rbuf)

    return pl.kernel(body, out_type=out_type, mesh=_sc_mesh(),
                     scratch_types=scratch, compiler_params=_SC_PARAMS)


def _make_deg():
    kd = 8
    cpt = TCH // (NC * NS)
    nss = cpt // kd
    out_type = jax.ShapeDtypeStruct((NC, NR, DG), F32)
    scratch = [
        pltpu.VMEM((CH, DG), F32),
        pltpu.VMEM((kd, CH), jnp.int32),
        pltpu.VMEM((CH, DG), F32),
        pltpu.VMEM_SHARED((NR, DG), F32),
        pltpu.SemaphoreType.DMA,
    ]

    def body(dst2, out, ones, didx, rbuf, acc, ssem):
        c = lax.axis_index("c")
        s = lax.axis_index("s")
        tile_r0 = s * RPT
        _zero_rows(rbuf)

        def zchunk(k, _):
            pltpu.sync_copy(rbuf, acc.at[pl.ds(tile_r0 + k * CH, CH)])
            return 0
        lax.fori_loop(0, DCH, zchunk, 0)

        def orow(r, _):
            ones[r, pl.ds(0, 16)] = jnp.ones((16,), F32)
            return 0
        lax.fori_loop(0, CH, orow, 0)
        plsc.subcore_barrier()

        base = (c * NS + s) * cpt

        def sstep(ss, _):
            row0 = base + ss * kd
            pltpu.sync_copy(dst2.at[pl.ds(row0, kd)], didx)
            sds = [pltpu.async_copy(ones, acc.at[didx.at[j]], ssem, add=True)
                   for j in range(kd)]
            for dd in sds:
                dd.wait()
            return 0
        lax.fori_loop(0, nss, sstep, 0)
        plsc.subcore_barrier()

        def dchunk(k, _):
            r0 = tile_r0 + k * CH
            pltpu.sync_copy(acc.at[pl.ds(r0, CH)], rbuf)
            pltpu.sync_copy(rbuf, out.at[c, pl.ds(r0, CH)])
            return 0
        lax.fori_loop(0, DCH, dchunk, 0)

    return pl.kernel(body, out_type=out_type, mesh=_sc_mesh(),
                     scratch_types=scratch, compiler_params=_SC_PARAMS)


_BN = 2048


def _prep_body(dref, xref, nref, n2ref, xsref, gref):
    dsum = dref[0, :, 0:1] + dref[1, :, 0:1]
    nrm = lax.rsqrt(jnp.maximum(dsum, 1.0))
    nref[...] = nrm
    n2ref[...] = nrm * nrm
    xb = xref[...]
    xsref[0] = xb[:, :16]
    xsref[1] = xb[:, 16:]
    gb = xb * nrm
    gref[0] = gb[:, :16]
    gref[1] = gb[:, 16:]


def _prep_call(deg_part, x_pad):
    grid = (NR // _BN,)
    return pl.pallas_call(
        _prep_body,
        grid=grid,
        in_specs=[
            pl.BlockSpec((2, _BN, DG), lambda i: (0, i, 0)),
            pl.BlockSpec((_BN, 32), lambda i: (i, 0)),
        ],
        out_specs=[
            pl.BlockSpec((_BN, 1), lambda i: (i, 0)),
            pl.BlockSpec((_BN, 1), lambda i: (i, 0)),
            pl.BlockSpec((2, _BN, DG), lambda i: (0, i, 0)),
            pl.BlockSpec((2, _BN, DG), lambda i: (0, i, 0)),
        ],
        out_shape=[
            jax.ShapeDtypeStruct((NR, 1), F32),
            jax.ShapeDtypeStruct((NR, 1), F32),
            jax.ShapeDtypeStruct((2, NR, DG), F32),
            jax.ShapeDtypeStruct((2, NR, DG), F32),
        ],
    )(deg_part, x_pad)


def _make_mm(ng, last):
    def body(href, s1ref, s2ref, wref, bref, nref, *orefs):
        n = nref[...]
        a = bref[...] + jnp.zeros((_BN, 64), F32)
        for g in range(ng):
            a = a + jnp.dot(href[g], wref[g], preferred_element_type=F32)
        for g in range(ng):
            a = a + jnp.dot(s1ref[g] * n, wref[ng + g],
                            preferred_element_type=F32)
        for g in range(ng):
            a = a + jnp.dot(s2ref[g] * n, wref[2 * ng + g],
                            preferred_element_type=F32)
        h = jnp.maximum(a, 0.0)
        for g in range(4):
            orefs[0][g] = h[:, g * DG:(g + 1) * DG]
        if not last:
            hn = h * n
            for g in range(4):
                orefs[1][g] = hn[:, g * DG:(g + 1) * DG]

    grid = (NR // _BN,)
    out_specs = [pl.BlockSpec((4, _BN, DG), lambda i: (0, i, 0))]
    out_shape = [jax.ShapeDtypeStruct((4, NR, DG), F32)]
    if not last:
        out_specs.append(pl.BlockSpec((4, _BN, DG), lambda i: (0, i, 0)))
        out_shape.append(jax.ShapeDtypeStruct((4, NR, DG), F32))

    def call(h_split, s1, s2, wb, bias, norm):
        return pl.pallas_call(
            body,
            grid=grid,
            in_specs=[
                pl.BlockSpec((ng, _BN, DG), lambda i: (0, i, 0)),
                pl.BlockSpec((ng, _BN, DG), lambda i: (0, i, 0)),
                pl.BlockSpec((ng, _BN, DG), lambda i: (0, i, 0)),
                pl.BlockSpec((3 * ng, DG, 64), lambda i: (0, 0, 0)),
                pl.BlockSpec((1, 64), lambda i: (0, 0)),
                pl.BlockSpec((_BN, 1), lambda i: (i, 0)),
            ],
            out_specs=out_specs,
            out_shape=out_shape,
        )(h_split, s1, s2, wb, bias, norm)
    return call


_PBN = 2000


def _pool_body(href, gref, wgref, bgref, oref, gmax_s, den_s, acc_s):
    p = pl.program_id(0)
    i = pl.program_id(1)
    nblk = pl.num_programs(1)
    gate = bgref[...] + jnp.zeros((_PBN, 1), F32)
    for g in range(4):
        gate = gate + jnp.dot(href[g], wgref[g], preferred_element_type=F32)
    ids = gref[...]
    onehot = ids == lax.broadcasted_iota(jnp.int32, (1, GG), 1)

    @pl.when(jnp.logical_and(p == 0, i == 0))
    def _():
        gmax_s[...] = jnp.full((1, GG), -jnp.inf, F32)

    @pl.when(p == 0)
    def _():
        m = jnp.max(jnp.where(onehot, gate, -jnp.inf), axis=0, keepdims=True)
        gmax_s[...] = jnp.maximum(gmax_s[...], m)

    @pl.when(p == 1)
    def _():
        @pl.when(i == 0)
        def _():
            den_s[...] = jnp.zeros((GG, 1), F32)
            acc_s[...] = jnp.zeros((GG, 64), F32)

        gm = gmax_s[...]
        gm = jnp.where(jnp.isfinite(gm), gm, 0.0)
        oh = onehot.astype(F32)
        gnode = lax.dot_general(oh, gm, (((1,), (1,)), ((), ())),
                                preferred_element_type=F32)
        e = jnp.exp(gate - gnode)
        den_s[...] = den_s[...] + lax.dot_general(
            oh, e, (((0,), (0,)), ((), ())), preferred_element_type=F32)
        h = jnp.concatenate([href[0], href[1], href[2], href[3]], axis=1)
        acc_s[...] = acc_s[...] + lax.dot_general(
            oh, e * h, (((0,), (0,)), ((), ())), preferred_element_type=F32)

        @pl.when(i == nblk - 1)
        def _():
            oref[...] = acc_s[...] / jnp.maximum(den_s[...], 1e-12)


def _pool_call(h3s, gid, wg4, bg2):
    grid = (2, NN // _PBN)
    return pl.pallas_call(
        _pool_body,
        grid=grid,
        in_specs=[
            pl.BlockSpec((4, _PBN, DG), lambda p, i: (0, i, 0)),
            pl.BlockSpec((_PBN, 1), lambda p, i: (i, 0)),
            pl.BlockSpec((4, DG, 1), lambda p, i: (0, 0, 0)),
            pl.BlockSpec((1, 1), lambda p, i: (0, 0)),
        ],
        out_specs=pl.BlockSpec((GG, 64), lambda p, i: (0, 0)),
        out_shape=jax.ShapeDtypeStruct((GG, 64), F32),
        scratch_shapes=[
            pltpu.VMEM((1, GG), F32),
            pltpu.VMEM((GG, 1), F32),
            pltpu.VMEM((GG, 64), F32),
        ],
    )(h3s, gid, wg4, bg2)


@jax.jit
def kernel(x, edge_index, graph_ids, W0, b0, W1, b1, W2, b2, W3, b3, Wg, bg):
    src = edge_index[0]
    dst = edge_index[1]
    epad = EP - EE
    src_pad = jnp.concatenate([src, jnp.zeros((epad,), jnp.int32)])
    dst_pad = jnp.concatenate([dst, jnp.full((epad,), NN, jnp.int32)])
    srcg = (src_pad[None, :]
            + (jnp.arange(4, dtype=jnp.int32) * NR)[:, None]).reshape(
                4, TCH, CH)
    dst2 = dst_pad.reshape(TCH, CH)
    x_pad = jnp.pad(x, ((0, NR - NN), (0, 1)))

    ridx = jnp.concatenate([jnp.arange(0, 31), jnp.arange(32, 63),
                            jnp.arange(64, 95)])
    wb0 = jnp.zeros((96, 64), F32).at[ridx].set(W0).reshape(6, DG, 64)
    wbs = [wb0, W1.reshape(12, DG, 64), W2.reshape(12, DG, 64),
           W3.reshape(12, DG, 64)]
    biases = [b0.reshape(1, 64), b1.reshape(1, 64), b2.reshape(1, 64),
              b3.reshape(1, 64)]

    deg_part = _make_deg()(dst2)
    norm, norm2, h_split, g = _prep_call(deg_part, x_pad)
    norm2f = norm2.reshape(NR)

    prop2_2 = _make_prop(2, True)
    prop1_2 = _make_prop(2, False)
    prop2_4 = _make_prop(4, True)
    prop1_4 = _make_prop(4, False)
    mm0 = _make_mm(2, False)
    mm = _make_mm(4, False)
    mm_last = _make_mm(4, True)

    ng = 2
    for layer in range(4):
        p2 = prop2_2 if ng == 2 else prop2_4
        p1 = prop1_2 if ng == 2 else prop1_4
        s1, g2 = p2(g.reshape(ng * NR, DG), srcg[:ng], dst2, norm2f)
        (s2,) = p1(g2.reshape(ng * NR, DG), srcg[:ng], dst2)
        if layer == 0:
            h_split, g = mm0(h_split, s1, s2, wbs[0], biases[0], norm)
        elif layer < 3:
            h_split, g = mm(h_split, s1, s2, wbs[layer], biases[layer], norm)
        else:
            (h_split,) = mm_last(h_split, s1, s2, wbs[3], biases[3], norm)
        ng = 4

    out = _pool_call(h_split, graph_ids.reshape(NN, 1),
                     Wg.reshape(4, DG, 1), bg.reshape(1, 1))
    return out.reshape(10, 50, 64)


# speedup vs baseline: 5.6471x; 1.1393x over previous
import jax
import jax.numpy as jnp
from jax import lax
from jax.experimental import pallas as pl
from jax.experimental.pallas import tpu as pltpu
from jax.experimental.pallas import tpu_sc as plsc

NN = 50000
NR = 51200
NC = 2
NS = 16
CH = 128
KF = 25
EE = 800000
EP = 819200
SSP = EP // (NS * CH * KF)
TCH = EP // CH
RPT = NR // NS
DCH = RPT // CH
GG = 500
DG = 16
F32 = jnp.float32

_SC_PARAMS = pltpu.CompilerParams(use_tc_tiling_on_sc=False)


def _sc_mesh():
    return plsc.VectorSubcoreMesh(core_axis_name="c", subcore_axis_name="s")


def _zero_rows(buf):
    def zrow(r, _):
        buf[r, pl.ds(0, 16)] = jnp.zeros((16,), F32)
        return 0
    lax.fori_loop(0, CH, zrow, 0)


def _make_prop(ngrp, scaled_out):
    ph = ngrp // NC
    n_out = 2 if scaled_out else 1
    out_type = [jax.ShapeDtypeStruct((ngrp, NR, DG), F32)
                for _ in range(n_out)]
    scratch = [
        pltpu.VMEM((KF, CH), jnp.int32),
        pltpu.VMEM((KF, CH), jnp.int32),
        pltpu.VMEM((KF, CH, DG), F32),
        pltpu.VMEM((CH, DG), F32),
        pltpu.VMEM((CH, DG), F32),
        pltpu.VMEM((CH,), F32),
        pltpu.VMEM_SHARED((NR, DG), F32),
        pltpu.SemaphoreType.DMA,
        pltpu.SemaphoreType.DMA,
    ]

    def body(*refs):
        if scaled_out:
            table, srcg, dst2, scale_h = refs[:4]
            outs = refs[4:4 + n_out]
            sidx, didx, gbuf, rbuf, obuf, scv, acc, gsem, ssem = refs[4 + n_out:]
        else:
            table, srcg, dst2 = refs[:3]
            outs = refs[3:3 + n_out]
            sidx, didx, gbuf, rbuf, obuf, scv, acc, gsem, ssem = refs[3 + n_out:]
        out_raw = outs[0]
        c = lax.axis_index("c")
        s = lax.axis_index("s")
        tile_r0 = s * RPT
        chunk0 = s * (SSP * KF)
        _zero_rows(rbuf)

        for p in range(ph):
            g = c * ph + p

            def zchunk(k, _):
                pltpu.sync_copy(rbuf, acc.at[pl.ds(tile_r0 + k * CH, CH)])
                return 0
            lax.fori_loop(0, DCH, zchunk, 0)
            plsc.subcore_barrier()

            def sstep(ss, _):
                row0 = chunk0 + ss * KF
                pltpu.sync_copy(srcg.at[g, pl.ds(row0, KF)], sidx)
                pltpu.sync_copy(dst2.at[pl.ds(row0, KF)], didx)
                gds = [pltpu.async_copy(table.at[sidx.at[j]], gbuf.at[j],
                                        gsem)
                       for j in range(KF)]
                sds = []
                for j in range(KF):
                    gds[j].wait()
                    sds.append(pltpu.async_copy(gbuf.at[j],
                                                acc.at[didx.at[j]], ssem,
                                                add=True))
                for dd in sds:
                    dd.wait()
                return 0
            lax.fori_loop(0, SSP, sstep, 0)
            plsc.subcore_barrier()

            def dchunk(k, _):
                r0 = tile_r0 + k * CH
                pltpu.sync_copy(acc.at[pl.ds(r0, CH)], rbuf)
                pltpu.sync_copy(rbuf, out_raw.at[g, pl.ds(r0, CH)])
                if scaled_out:
                    pltpu.sync_copy(scale_h.at[pl.ds(r0, CH)], scv)

                    def sgrp(rg, _):
                        rr = rg * 16
                        sv16 = scv[pl.ds(rr, 16)]
                        for kk in range(16):
                            obuf[rr + kk, pl.ds(0, 16)] = (
                                rbuf[rr + kk, pl.ds(0, 16)] * sv16[kk])
                        return 0
                    lax.fori_loop(0, CH // 16, sgrp, 0)
                    pltpu.sync_copy(obuf, outs[1].at[g, pl.ds(r0, CH)])
                return 0
            lax.fori_loop(0, DCH, dchunk, 0)
            if p + 1 < ph:
                _zero_rows(rbuf)

    return pl.kernel(body, out_type=out_type, mesh=_sc_mesh(),
                     scratch_types=scratch, compiler_params=_SC_PARAMS)


def _make_deg():
    kd = 25
    cpt = TCH // (NC * NS)
    nss = cpt // kd
    out_type = jax.ShapeDtypeStruct((NC, NR, DG), F32)
    scratch = [
        pltpu.VMEM((CH, DG), F32),
        pltpu.VMEM((kd, CH), jnp.int32),
        pltpu.VMEM((CH, DG), F32),
        pltpu.VMEM_SHARED((NR, DG), F32),
        pltpu.SemaphoreType.DMA,
    ]

    def body(dst2, out, ones, didx, rbuf, acc, ssem):
        c = lax.axis_index("c")
        s = lax.axis_index("s")
        tile_r0 = s * RPT
        _zero_rows(rbuf)

        def zchunk(k, _):
            pltpu.sync_copy(rbuf, acc.at[pl.ds(tile_r0 + k * CH, CH)])
            return 0
        lax.fori_loop(0, DCH, zchunk, 0)

        def orow(r, _):
            ones[r, pl.ds(0, 16)] = jnp.ones((16,), F32)
            return 0
        lax.fori_loop(0, CH, orow, 0)
        plsc.subcore_barrier()

        base = (c * NS + s) * cpt

        def sstep(ss, _):
            row0 = base + ss * kd
            pltpu.sync_copy(dst2.at[pl.ds(row0, kd)], didx)
            sds = [pltpu.async_copy(ones, acc.at[didx.at[j]], ssem, add=True)
                   for j in range(kd)]
            for dd in sds:
                dd.wait()
            return 0
        lax.fori_loop(0, nss, sstep, 0)
        plsc.subcore_barrier()

        def dchunk(k, _):
            r0 = tile_r0 + k * CH
            pltpu.sync_copy(acc.at[pl.ds(r0, CH)], rbuf)
            pltpu.sync_copy(rbuf, out.at[c, pl.ds(r0, CH)])
            return 0
        lax.fori_loop(0, DCH, dchunk, 0)

    return pl.kernel(body, out_type=out_type, mesh=_sc_mesh(),
                     scratch_types=scratch, compiler_params=_SC_PARAMS)


_BN = 2048


def _prep_body(dref, xref, nref, n2ref, xsref, gref):
    dsum = dref[0, :, 0:1] + dref[1, :, 0:1]
    nrm = lax.rsqrt(jnp.maximum(dsum, 1.0))
    nref[...] = nrm
    n2ref[...] = nrm * nrm
    xb = xref[...]
    xsref[0] = xb[:, :16]
    xsref[1] = xb[:, 16:]
    gb = xb * nrm
    gref[0] = gb[:, :16]
    gref[1] = gb[:, 16:]


def _prep_call(deg_part, x_pad):
    grid = (NR // _BN,)
    return pl.pallas_call(
        _prep_body,
        grid=grid,
        in_specs=[
            pl.BlockSpec((2, _BN, DG), lambda i: (0, i, 0)),
            pl.BlockSpec((_BN, 32), lambda i: (i, 0)),
        ],
        out_specs=[
            pl.BlockSpec((_BN, 1), lambda i: (i, 0)),
            pl.BlockSpec((_BN, 1), lambda i: (i, 0)),
            pl.BlockSpec((2, _BN, DG), lambda i: (0, i, 0)),
            pl.BlockSpec((2, _BN, DG), lambda i: (0, i, 0)),
        ],
        out_shape=[
            jax.ShapeDtypeStruct((NR, 1), F32),
            jax.ShapeDtypeStruct((NR, 1), F32),
            jax.ShapeDtypeStruct((2, NR, DG), F32),
            jax.ShapeDtypeStruct((2, NR, DG), F32),
        ],
    )(deg_part, x_pad)


def _make_mm(ng, last):
    def body(href, s1ref, s2ref, wref, bref, nref, *orefs):
        n = nref[...]
        a = bref[...] + jnp.zeros((_BN, 64), F32)
        for g in range(ng):
            a = a + jnp.dot(href[g], wref[g], preferred_element_type=F32)
        for g in range(ng):
            a = a + jnp.dot(s1ref[g] * n, wref[ng + g],
                            preferred_element_type=F32)
        for g in range(ng):
            a = a + jnp.dot(s2ref[g] * n, wref[2 * ng + g],
                            preferred_element_type=F32)
        h = jnp.maximum(a, 0.0)
        for g in range(4):
            orefs[0][g] = h[:, g * DG:(g + 1) * DG]
        if not last:
            hn = h * n
            for g in range(4):
                orefs[1][g] = hn[:, g * DG:(g + 1) * DG]

    grid = (NR // _BN,)
    out_specs = [pl.BlockSpec((4, _BN, DG), lambda i: (0, i, 0))]
    out_shape = [jax.ShapeDtypeStruct((4, NR, DG), F32)]
    if not last:
        out_specs.append(pl.BlockSpec((4, _BN, DG), lambda i: (0, i, 0)))
        out_shape.append(jax.ShapeDtypeStruct((4, NR, DG), F32))

    def call(h_split, s1, s2, wb, bias, norm):
        return pl.pallas_call(
            body,
            grid=grid,
            in_specs=[
                pl.BlockSpec((ng, _BN, DG), lambda i: (0, i, 0)),
                pl.BlockSpec((ng, _BN, DG), lambda i: (0, i, 0)),
                pl.BlockSpec((ng, _BN, DG), lambda i: (0, i, 0)),
                pl.BlockSpec((3 * ng, DG, 64), lambda i: (0, 0, 0)),
                pl.BlockSpec((1, 64), lambda i: (0, 0)),
                pl.BlockSpec((_BN, 1), lambda i: (i, 0)),
            ],
            out_specs=out_specs,
            out_shape=out_shape,
        )(h_split, s1, s2, wb, bias, norm)
    return call


_PBN = 2000


def _pool_body(href, gref, wgref, bgref, oref, gmax_s, den_s, acc_s):
    p = pl.program_id(0)
    i = pl.program_id(1)
    nblk = pl.num_programs(1)
    gate = bgref[...] + jnp.zeros((_PBN, 1), F32)
    for g in range(4):
        gate = gate + jnp.dot(href[g], wgref[g], preferred_element_type=F32)
    ids = gref[...]
    onehot = ids == lax.broadcasted_iota(jnp.int32, (1, GG), 1)

    @pl.when(jnp.logical_and(p == 0, i == 0))
    def _():
        gmax_s[...] = jnp.full((1, GG), -jnp.inf, F32)

    @pl.when(p == 0)
    def _():
        m = jnp.max(jnp.where(onehot, gate, -jnp.inf), axis=0, keepdims=True)
        gmax_s[...] = jnp.maximum(gmax_s[...], m)

    @pl.when(p == 1)
    def _():
        @pl.when(i == 0)
        def _():
            den_s[...] = jnp.zeros((GG, 1), F32)
            acc_s[...] = jnp.zeros((GG, 64), F32)

        gm = gmax_s[...]
        gm = jnp.where(jnp.isfinite(gm), gm, 0.0)
        oh = onehot.astype(F32)
        gnode = lax.dot_general(oh, gm, (((1,), (1,)), ((), ())),
                                preferred_element_type=F32)
        e = jnp.exp(gate - gnode)
        den_s[...] = den_s[...] + lax.dot_general(
            oh, e, (((0,), (0,)), ((), ())), preferred_element_type=F32)
        h = jnp.concatenate([href[0], href[1], href[2], href[3]], axis=1)
        acc_s[...] = acc_s[...] + lax.dot_general(
            oh, e * h, (((0,), (0,)), ((), ())), preferred_element_type=F32)

        @pl.when(i == nblk - 1)
        def _():
            oref[...] = acc_s[...] / jnp.maximum(den_s[...], 1e-12)


def _pool_call(h3s, gid, wg4, bg2):
    grid = (2, NN // _PBN)
    return pl.pallas_call(
        _pool_body,
        grid=grid,
        in_specs=[
            pl.BlockSpec((4, _PBN, DG), lambda p, i: (0, i, 0)),
            pl.BlockSpec((_PBN, 1), lambda p, i: (i, 0)),
            pl.BlockSpec((4, DG, 1), lambda p, i: (0, 0, 0)),
            pl.BlockSpec((1, 1), lambda p, i: (0, 0)),
        ],
        out_specs=pl.BlockSpec((GG, 64), lambda p, i: (0, 0)),
        out_shape=jax.ShapeDtypeStruct((GG, 64), F32),
        scratch_shapes=[
            pltpu.VMEM((1, GG), F32),
            pltpu.VMEM((GG, 1), F32),
            pltpu.VMEM((GG, 64), F32),
        ],
    )(h3s, gid, wg4, bg2)


@jax.jit
def kernel(x, edge_index, graph_ids, W0, b0, W1, b1, W2, b2, W3, b3, Wg, bg):
    src = edge_index[0]
    dst = edge_index[1]
    epad = EP - EE
    src_pad = jnp.concatenate([src, jnp.zeros((epad,), jnp.int32)])
    dst_pad = jnp.concatenate([dst, jnp.full((epad,), NN, jnp.int32)])
    srcg = (src_pad[None, :]
            + (jnp.arange(4, dtype=jnp.int32) * NR)[:, None]).reshape(
                4, TCH, CH)
    dst2 = dst_pad.reshape(TCH, CH)
    x_pad = jnp.pad(x, ((0, NR - NN), (0, 1)))

    ridx = jnp.concatenate([jnp.arange(0, 31), jnp.arange(32, 63),
                            jnp.arange(64, 95)])
    wb0 = jnp.zeros((96, 64), F32).at[ridx].set(W0).reshape(6, DG, 64)
    wbs = [wb0, W1.reshape(12, DG, 64), W2.reshape(12, DG, 64),
           W3.reshape(12, DG, 64)]
    biases = [b0.reshape(1, 64), b1.reshape(1, 64), b2.reshape(1, 64),
              b3.reshape(1, 64)]

    deg_part = _make_deg()(dst2)
    norm, norm2, h_split, g = _prep_call(deg_part, x_pad)
    norm2f = norm2.reshape(NR)

    prop2_2 = _make_prop(2, True)
    prop1_2 = _make_prop(2, False)
    prop2_4 = _make_prop(4, True)
    prop1_4 = _make_prop(4, False)
    mm0 = _make_mm(2, False)
    mm = _make_mm(4, False)
    mm_last = _make_mm(4, True)

    ng = 2
    for layer in range(4):
        p2 = prop2_2 if ng == 2 else prop2_4
        p1 = prop1_2 if ng == 2 else prop1_4
        s1, g2 = p2(g.reshape(ng * NR, DG), srcg[:ng], dst2, norm2f)
        (s2,) = p1(g2.reshape(ng * NR, DG), srcg[:ng], dst2)
        if layer == 0:
            h_split, g = mm0(h_split, s1, s2, wbs[0], biases[0], norm)
        elif layer < 3:
            h_split, g = mm(h_split, s1, s2, wbs[layer], biases[layer], norm)
        else:
            (h_split,) = mm_last(h_split, s1, s2, wbs[3], biases[3], norm)
        ng = 4

    out = _pool_call(h_split, graph_ids.reshape(NN, 1),
                     Wg.reshape(4, DG, 1), bg.reshape(1, 1))
    return out.reshape(10, 50, 64)


# speedup vs baseline: 5.9674x; 1.0567x over previous
import jax
import jax.numpy as jnp
from jax import lax
from jax.experimental import pallas as pl
from jax.experimental.pallas import tpu as pltpu
from jax.experimental.pallas import tpu_sc as plsc

NN = 50000
NR = 51200
NC = 2
NS = 16
CH = 128
KF = 25
EE = 800000
EP = 819200
SSP = EP // (NS * CH * KF)
TCH = EP // CH
RPT = NR // NS
DCH = RPT // CH
GG = 500
DG = 16
F32 = jnp.float32

_SC_PARAMS = pltpu.CompilerParams(use_tc_tiling_on_sc=False)


def _sc_mesh():
    return plsc.VectorSubcoreMesh(core_axis_name="c", subcore_axis_name="s")


def _zero_rows(buf):
    def zrow(r, _):
        buf[r, pl.ds(0, 16)] = jnp.zeros((16,), F32)
        return 0
    lax.fori_loop(0, CH, zrow, 0)


def _make_prop(ngrp, scaled_out):
    ph = ngrp // NC
    n_out = 2 if scaled_out else 1
    out_type = [jax.ShapeDtypeStruct((ngrp, NR, DG), F32)
                for _ in range(n_out)]
    scratch = [
        pltpu.VMEM((KF, CH), jnp.int32),
        pltpu.VMEM((KF, CH), jnp.int32),
        pltpu.VMEM((KF, CH, DG), F32),
        pltpu.VMEM((2, CH, DG), F32),
        pltpu.VMEM((2, CH, DG), F32),
        pltpu.VMEM((RPT,), F32),
        pltpu.VMEM_SHARED((NR, DG), F32),
        pltpu.SemaphoreType.DMA,
        pltpu.SemaphoreType.DMA,
        pltpu.SemaphoreType.DMA,
        pltpu.SemaphoreType.DMA,
    ]

    def body(*refs):
        if scaled_out:
            table, srcg, dst2, scale_h = refs[:4]
            outs = refs[4:4 + n_out]
            rest = refs[4 + n_out:]
        else:
            table, srcg, dst2 = refs[:3]
            outs = refs[3:3 + n_out]
            rest = refs[3 + n_out:]
        sidx, didx, gbuf, rbuf, obuf, scv, acc, gsem, ssem, rsem, wsem = rest
        out_raw = outs[0]
        c = lax.axis_index("c")
        s = lax.axis_index("s")
        tile_r0 = s * RPT
        chunk0 = s * (SSP * KF)
        _zero_rows(rbuf.at[0])

        def scale_chunk(k, b):
            def sgrp(rg, _):
                rr = rg * 16
                sv16 = scv[pl.ds(k * CH + rr, 16)]
                for kk in range(16):
                    obuf[b, rr + kk, pl.ds(0, 16)] = (
                        rbuf[b, rr + kk, pl.ds(0, 16)] * sv16[kk])
                return 0
            lax.fori_loop(0, CH // 16, sgrp, 0)

        for p in range(ph):
            g = c * ph + p

            zds = [pltpu.async_copy(rbuf.at[0],
                                    acc.at[pl.ds(tile_r0 + k * CH, CH)],
                                    wsem)
                   for k in range(DCH)]
            if scaled_out:
                pltpu.sync_copy(scale_h.at[pl.ds(tile_r0, RPT)], scv)
            for dd in zds:
                dd.wait()
            plsc.subcore_barrier()

            def sstep(ss, _):
                row0 = chunk0 + ss * KF
                ia = pltpu.async_copy(srcg.at[g, pl.ds(row0, KF)], sidx,
                                      rsem)
                ib = pltpu.async_copy(dst2.at[pl.ds(row0, KF)], didx, rsem)
                ia.wait()
                ib.wait()
                gds = [pltpu.async_copy(table.at[sidx.at[j]], gbuf.at[j],
                                        gsem)
                       for j in range(KF)]
                sds = []
                for j in range(KF):
                    gds[j].wait()
                    sds.append(pltpu.async_copy(gbuf.at[j],
                                                acc.at[didx.at[j]], ssem,
                                                add=True))
                for dd in sds:
                    dd.wait()
                return 0
            lax.fori_loop(0, SSP, sstep, 0)
            plsc.subcore_barrier()

            def read_chunk(k, b):
                return pltpu.async_copy(acc.at[pl.ds(tile_r0 + k * CH, CH)],
                                        rbuf.at[b], rsem)

            def write_chunk(k, b):
                wl = [pltpu.async_copy(rbuf.at[b],
                                       out_raw.at[g, pl.ds(tile_r0 + k * CH,
                                                           CH)],
                                       wsem)]
                if scaled_out:
                    scale_chunk(k, b)
                    wl.append(pltpu.async_copy(
                        obuf.at[b],
                        outs[1].at[g, pl.ds(tile_r0 + k * CH, CH)], wsem))
                return wl

            rd = read_chunk(0, 0)
            pend = []
            for k in range(DCH):
                b = k % 2
                rd.wait()
                for dd in pend:
                    dd.wait()
                if k + 1 < DCH:
                    rd = read_chunk(k + 1, 1 - b)
                pend = write_chunk(k, b)
            for dd in pend:
                dd.wait()
            if p + 1 < ph:
                _zero_rows(rbuf.at[0])

    return pl.kernel(body, out_type=out_type, mesh=_sc_mesh(),
                     scratch_types=scratch, compiler_params=_SC_PARAMS)


def _make_deg():
    kd = 25
    cpt = TCH // (NC * NS)
    nss = cpt // kd
    out_type = jax.ShapeDtypeStruct((NC, NR, DG), F32)
    scratch = [
        pltpu.VMEM((CH, DG), F32),
        pltpu.VMEM((kd, CH), jnp.int32),
        pltpu.VMEM((2, CH, DG), F32),
        pltpu.VMEM_SHARED((NR, DG), F32),
        pltpu.SemaphoreType.DMA,
        pltpu.SemaphoreType.DMA,
        pltpu.SemaphoreType.DMA,
    ]

    def body(dst2, out, ones, didx, rbuf, acc, ssem, rsem, wsem):
        c = lax.axis_index("c")
        s = lax.axis_index("s")
        tile_r0 = s * RPT
        _zero_rows(rbuf.at[0])
        zds = [pltpu.async_copy(rbuf.at[0],
                                acc.at[pl.ds(tile_r0 + k * CH, CH)], wsem)
               for k in range(DCH)]

        def orow(r, _):
            ones[r, pl.ds(0, 16)] = jnp.ones((16,), F32)
            return 0
        lax.fori_loop(0, CH, orow, 0)
        for dd in zds:
            dd.wait()
        plsc.subcore_barrier()

        base = (c * NS + s) * cpt

        def sstep(ss, _):
            row0 = base + ss * kd
            pltpu.async_copy(dst2.at[pl.ds(row0, kd)], didx, rsem).wait()
            sds = [pltpu.async_copy(ones, acc.at[didx.at[j]], ssem, add=True)
                   for j in range(kd)]
            for dd in sds:
                dd.wait()
            return 0
        lax.fori_loop(0, nss, sstep, 0)
        plsc.subcore_barrier()

        def read_chunk(k, b):
            return pltpu.async_copy(acc.at[pl.ds(tile_r0 + k * CH, CH)],
                                    rbuf.at[b], rsem)

        rd = read_chunk(0, 0)
        pend = []
        for k in range(DCH):
            b = k % 2
            rd.wait()
            for dd in pend:
                dd.wait()
            if k + 1 < DCH:
                rd = read_chunk(k + 1, 1 - b)
            pend = [pltpu.async_copy(rbuf.at[b],
                                     out.at[c, pl.ds(tile_r0 + k * CH, CH)],
                                     wsem)]
        for dd in pend:
            dd.wait()

    return pl.kernel(body, out_type=out_type, mesh=_sc_mesh(),
                     scratch_types=scratch, compiler_params=_SC_PARAMS)


_BN = 2048


def _prep_body(dref, xref, nref, n2ref, xsref, gref):
    dsum = dref[0, :, 0:1] + dref[1, :, 0:1]
    nrm = lax.rsqrt(jnp.maximum(dsum, 1.0))
    nref[...] = nrm
    n2ref[...] = nrm * nrm
    xb = xref[...]
    xsref[0] = xb[:, :16]
    xsref[1] = xb[:, 16:]
    gb = xb * nrm
    gref[0] = gb[:, :16]
    gref[1] = gb[:, 16:]


def _prep_call(deg_part, x_pad):
    grid = (NR // _BN,)
    return pl.pallas_call(
        _prep_body,
        grid=grid,
        in_specs=[
            pl.BlockSpec((2, _BN, DG), lambda i: (0, i, 0)),
            pl.BlockSpec((_BN, 32), lambda i: (i, 0)),
        ],
        out_specs=[
            pl.BlockSpec((_BN, 1), lambda i: (i, 0)),
            pl.BlockSpec((_BN, 1), lambda i: (i, 0)),
            pl.BlockSpec((2, _BN, DG), lambda i: (0, i, 0)),
            pl.BlockSpec((2, _BN, DG), lambda i: (0, i, 0)),
        ],
        out_shape=[
            jax.ShapeDtypeStruct((NR, 1), F32),
            jax.ShapeDtypeStruct((NR, 1), F32),
            jax.ShapeDtypeStruct((2, NR, DG), F32),
            jax.ShapeDtypeStruct((2, NR, DG), F32),
        ],
    )(deg_part, x_pad)


def _make_mm(ng, last):
    def body(href, s1ref, s2ref, wref, bref, nref, *orefs):
        n = nref[...]
        a = bref[...] + jnp.zeros((_BN, 64), F32)
        for g in range(ng):
            a = a + jnp.dot(href[g], wref[g], preferred_element_type=F32)
        for g in range(ng):
            a = a + jnp.dot(s1ref[g] * n, wref[ng + g],
                            preferred_element_type=F32)
        for g in range(ng):
            a = a + jnp.dot(s2ref[g] * n, wref[2 * ng + g],
                            preferred_element_type=F32)
        h = jnp.maximum(a, 0.0)
        for g in range(4):
            orefs[0][g] = h[:, g * DG:(g + 1) * DG]
        if not last:
            hn = h * n
            for g in range(4):
                orefs[1][g] = hn[:, g * DG:(g + 1) * DG]

    grid = (NR // _BN,)
    out_specs = [pl.BlockSpec((4, _BN, DG), lambda i: (0, i, 0))]
    out_shape = [jax.ShapeDtypeStruct((4, NR, DG), F32)]
    if not last:
        out_specs.append(pl.BlockSpec((4, _BN, DG), lambda i: (0, i, 0)))
        out_shape.append(jax.ShapeDtypeStruct((4, NR, DG), F32))

    def call(h_split, s1, s2, wb, bias, norm):
        return pl.pallas_call(
            body,
            grid=grid,
            in_specs=[
                pl.BlockSpec((ng, _BN, DG), lambda i: (0, i, 0)),
                pl.BlockSpec((ng, _BN, DG), lambda i: (0, i, 0)),
                pl.BlockSpec((ng, _BN, DG), lambda i: (0, i, 0)),
                pl.BlockSpec((3 * ng, DG, 64), lambda i: (0, 0, 0)),
                pl.BlockSpec((1, 64), lambda i: (0, 0)),
                pl.BlockSpec((_BN, 1), lambda i: (i, 0)),
            ],
            out_specs=out_specs,
            out_shape=out_shape,
        )(h_split, s1, s2, wb, bias, norm)
    return call


_PBN = 2000


def _pool_body(href, gref, wgref, bgref, oref, gmax_s, den_s, acc_s):
    p = pl.program_id(0)
    i = pl.program_id(1)
    nblk = pl.num_programs(1)
    gate = bgref[...] + jnp.zeros((_PBN, 1), F32)
    for g in range(4):
        gate = gate + jnp.dot(href[g], wgref[g], preferred_element_type=F32)
    ids = gref[...]
    onehot = ids == lax.broadcasted_iota(jnp.int32, (1, GG), 1)

    @pl.when(jnp.logical_and(p == 0, i == 0))
    def _():
        gmax_s[...] = jnp.full((1, GG), -jnp.inf, F32)

    @pl.when(p == 0)
    def _():
        m = jnp.max(jnp.where(onehot, gate, -jnp.inf), axis=0, keepdims=True)
        gmax_s[...] = jnp.maximum(gmax_s[...], m)

    @pl.when(p == 1)
    def _():
        @pl.when(i == 0)
        def _():
            den_s[...] = jnp.zeros((GG, 1), F32)
            acc_s[...] = jnp.zeros((GG, 64), F32)

        gm = gmax_s[...]
        gm = jnp.where(jnp.isfinite(gm), gm, 0.0)
        oh = onehot.astype(F32)
        gnode = lax.dot_general(oh, gm, (((1,), (1,)), ((), ())),
                                preferred_element_type=F32)
        e = jnp.exp(gate - gnode)
        den_s[...] = den_s[...] + lax.dot_general(
            oh, e, (((0,), (0,)), ((), ())), preferred_element_type=F32)
        h = jnp.concatenate([href[0], href[1], href[2], href[3]], axis=1)
        acc_s[...] = acc_s[...] + lax.dot_general(
            oh, e * h, (((0,), (0,)), ((), ())), preferred_element_type=F32)

        @pl.when(i == nblk - 1)
        def _():
            oref[...] = acc_s[...] / jnp.maximum(den_s[...], 1e-12)


def _pool_call(h3s, gid, wg4, bg2):
    grid = (2, NN // _PBN)
    return pl.pallas_call(
        _pool_body,
        grid=grid,
        in_specs=[
            pl.BlockSpec((4, _PBN, DG), lambda p, i: (0, i, 0)),
            pl.BlockSpec((_PBN, 1), lambda p, i: (i, 0)),
            pl.BlockSpec((4, DG, 1), lambda p, i: (0, 0, 0)),
            pl.BlockSpec((1, 1), lambda p, i: (0, 0)),
        ],
        out_specs=pl.BlockSpec((GG, 64), lambda p, i: (0, 0)),
        out_shape=jax.ShapeDtypeStruct((GG, 64), F32),
        scratch_shapes=[
            pltpu.VMEM((1, GG), F32),
            pltpu.VMEM((GG, 1), F32),
            pltpu.VMEM((GG, 64), F32),
        ],
    )(h3s, gid, wg4, bg2)


@jax.jit
def kernel(x, edge_index, graph_ids, W0, b0, W1, b1, W2, b2, W3, b3, Wg, bg):
    src = edge_index[0]
    dst = edge_index[1]
    epad = EP - EE
    src_pad = jnp.concatenate([src, jnp.zeros((epad,), jnp.int32)])
    dst_pad = jnp.concatenate([dst, jnp.full((epad,), NN, jnp.int32)])
    srcg = (src_pad[None, :]
            + (jnp.arange(4, dtype=jnp.int32) * NR)[:, None]).reshape(
                4, TCH, CH)
    dst2 = dst_pad.reshape(TCH, CH)
    x_pad = jnp.pad(x, ((0, NR - NN), (0, 1)))

    ridx = jnp.concatenate([jnp.arange(0, 31), jnp.arange(32, 63),
                            jnp.arange(64, 95)])
    wb0 = jnp.zeros((96, 64), F32).at[ridx].set(W0).reshape(6, DG, 64)
    wbs = [wb0, W1.reshape(12, DG, 64), W2.reshape(12, DG, 64),
           W3.reshape(12, DG, 64)]
    biases = [b0.reshape(1, 64), b1.reshape(1, 64), b2.reshape(1, 64),
              b3.reshape(1, 64)]

    deg_part = _make_deg()(dst2)
    norm, norm2, h_split, g = _prep_call(deg_part, x_pad)
    norm2f = norm2.reshape(NR)

    prop2_2 = _make_prop(2, True)
    prop1_2 = _make_prop(2, False)
    prop2_4 = _make_prop(4, True)
    prop1_4 = _make_prop(4, False)
    mm0 = _make_mm(2, False)
    mm = _make_mm(4, False)
    mm_last = _make_mm(4, True)

    ng = 2
    for layer in range(4):
        p2 = prop2_2 if ng == 2 else prop2_4
        p1 = prop1_2 if ng == 2 else prop1_4
        s1, g2 = p2(g.reshape(ng * NR, DG), srcg[:ng], dst2, norm2f)
        (s2,) = p1(g2.reshape(ng * NR, DG), srcg[:ng], dst2)
        if layer == 0:
            h_split, g = mm0(h_split, s1, s2, wbs[0], biases[0], norm)
        elif layer < 3:
            h_split, g = mm(h_split, s1, s2, wbs[layer], biases[layer], norm)
        else:
            (h_split,) = mm_last(h_split, s1, s2, wbs[3], biases[3], norm)
        ng = 4

    out = _pool_call(h_split, graph_ids.reshape(NN, 1),
                     Wg.reshape(4, DG, 1), bg.reshape(1, 1))
    return out.reshape(10, 50, 64)


# speedup vs baseline: 6.1340x; 1.0279x over previous
import jax
import jax.numpy as jnp
from jax import lax
from jax.experimental import pallas as pl
from jax.experimental.pallas import tpu as pltpu
from jax.experimental.pallas import tpu_sc as plsc

NN = 50000
NR = 51200
NC = 2
NS = 16
CH = 128
KF = 25
EE = 800000
EP = 819200
SSP = EP // (NS * CH * KF)
TCH = EP // CH
RPT = NR // NS
DCH = RPT // CH
GG = 500
DG = 16
F32 = jnp.float32

_SC_PARAMS = pltpu.CompilerParams(use_tc_tiling_on_sc=False)


def _sc_mesh():
    return plsc.VectorSubcoreMesh(core_axis_name="c", subcore_axis_name="s")


def _zero_rows(buf):
    def zrow(r, _):
        buf[r, pl.ds(0, 16)] = jnp.zeros((16,), F32)
        return 0
    lax.fori_loop(0, CH, zrow, 0)


def _make_prop(ngrp, scaled_out):
    ph = ngrp // NC
    n_out = 2 if scaled_out else 1
    out_type = [jax.ShapeDtypeStruct((ngrp, NR, DG), F32)
                for _ in range(n_out)]
    scratch = [
        pltpu.VMEM((KF, CH), jnp.int32),
        pltpu.VMEM((KF, CH), jnp.int32),
        pltpu.VMEM((KF, CH, DG), F32),
        pltpu.VMEM((2, CH, DG), F32),
        pltpu.VMEM((2, CH, DG), F32),
        pltpu.VMEM((RPT,), F32),
        pltpu.VMEM_SHARED((NR, DG), F32),
        pltpu.SemaphoreType.DMA,
        pltpu.SemaphoreType.DMA,
        pltpu.SemaphoreType.DMA,
        pltpu.SemaphoreType.DMA,
    ]

    def body(*refs):
        if scaled_out:
            table, src2, dst2, scale_h = refs[:4]
            outs = refs[4:4 + n_out]
            rest = refs[4 + n_out:]
        else:
            table, src2, dst2 = refs[:3]
            outs = refs[3:3 + n_out]
            rest = refs[3 + n_out:]
        sidx, didx, gbuf, rbuf, obuf, scv, acc, gsem, ssem, rsem, wsem = rest
        out_raw = outs[0]
        c = lax.axis_index("c")
        s = lax.axis_index("s")
        tile_r0 = s * RPT
        chunk0 = s * (SSP * KF)
        _zero_rows(rbuf.at[0])

        def scale_chunk(k, b):
            def sgrp(rg, _):
                rr = rg * 16
                sv16 = scv[pl.ds(k * CH + rr, 16)]
                for kk in range(16):
                    obuf[b, rr + kk, pl.ds(0, 16)] = (
                        rbuf[b, rr + kk, pl.ds(0, 16)] * sv16[kk])
                return 0
            lax.fori_loop(0, CH // 16, sgrp, 0)

        for p in range(ph):
            g = c * ph + p

            zds = [pltpu.async_copy(rbuf.at[0],
                                    acc.at[pl.ds(tile_r0 + k * CH, CH)],
                                    wsem)
                   for k in range(DCH)]
            if scaled_out:
                pltpu.sync_copy(scale_h.at[pl.ds(tile_r0, RPT)], scv)
            for dd in zds:
                dd.wait()
            plsc.subcore_barrier()

            def sstep(ss, _):
                row0 = chunk0 + ss * KF
                ia = pltpu.async_copy(src2.at[pl.ds(row0, KF)], sidx,
                                      rsem)
                ib = pltpu.async_copy(dst2.at[pl.ds(row0, KF)], didx, rsem)
                ia.wait()
                ib.wait()
                gds = [pltpu.async_copy(table.at[g].at[sidx.at[j]],
                                        gbuf.at[j], gsem)
                       for j in range(KF)]
                sds = []
                for j in range(KF):
                    gds[j].wait()
                    sds.append(pltpu.async_copy(gbuf.at[j],
                                                acc.at[didx.at[j]], ssem,
                                                add=True))
                for dd in sds:
                    dd.wait()
                return 0
            lax.fori_loop(0, SSP, sstep, 0)
            plsc.subcore_barrier()

            def read_chunk(k, b):
                return pltpu.async_copy(acc.at[pl.ds(tile_r0 + k * CH, CH)],
                                        rbuf.at[b], rsem)

            def write_chunk(k, b):
                wl = [pltpu.async_copy(rbuf.at[b],
                                       out_raw.at[g, pl.ds(tile_r0 + k * CH,
                                                           CH)],
                                       wsem)]
                if scaled_out:
                    scale_chunk(k, b)
                    wl.append(pltpu.async_copy(
                        obuf.at[b],
                        outs[1].at[g, pl.ds(tile_r0 + k * CH, CH)], wsem))
                return wl

            rd = read_chunk(0, 0)
            pend = []
            for k in range(DCH):
                b = k % 2
                rd.wait()
                for dd in pend:
                    dd.wait()
                if k + 1 < DCH:
                    rd = read_chunk(k + 1, 1 - b)
                pend = write_chunk(k, b)
            for dd in pend:
                dd.wait()
            if p + 1 < ph:
                _zero_rows(rbuf.at[0])

    return pl.kernel(body, out_type=out_type, mesh=_sc_mesh(),
                     scratch_types=scratch, compiler_params=_SC_PARAMS)


def _make_deg():
    kd = 25
    cpt = TCH // (NC * NS)
    nss = cpt // kd
    out_type = jax.ShapeDtypeStruct((NC, NR, DG), F32)
    scratch = [
        pltpu.VMEM((CH, DG), F32),
        pltpu.VMEM((kd, CH), jnp.int32),
        pltpu.VMEM((2, CH, DG), F32),
        pltpu.VMEM_SHARED((NR, DG), F32),
        pltpu.SemaphoreType.DMA,
        pltpu.SemaphoreType.DMA,
        pltpu.SemaphoreType.DMA,
    ]

    def body(dst2, out, ones, didx, rbuf, acc, ssem, rsem, wsem):
        c = lax.axis_index("c")
        s = lax.axis_index("s")
        tile_r0 = s * RPT
        _zero_rows(rbuf.at[0])
        zds = [pltpu.async_copy(rbuf.at[0],
                                acc.at[pl.ds(tile_r0 + k * CH, CH)], wsem)
               for k in range(DCH)]

        def orow(r, _):
            ones[r, pl.ds(0, 16)] = jnp.ones((16,), F32)
            return 0
        lax.fori_loop(0, CH, orow, 0)
        for dd in zds:
            dd.wait()
        plsc.subcore_barrier()

        base = (c * NS + s) * cpt

        def sstep(ss, _):
            row0 = base + ss * kd
            pltpu.async_copy(dst2.at[pl.ds(row0, kd)], didx, rsem).wait()
            sds = [pltpu.async_copy(ones, acc.at[didx.at[j]], ssem, add=True)
                   for j in range(kd)]
            for dd in sds:
                dd.wait()
            return 0
        lax.fori_loop(0, nss, sstep, 0)
        plsc.subcore_barrier()

        def read_chunk(k, b):
            return pltpu.async_copy(acc.at[pl.ds(tile_r0 + k * CH, CH)],
                                    rbuf.at[b], rsem)

        rd = read_chunk(0, 0)
        pend = []
        for k in range(DCH):
            b = k % 2
            rd.wait()
            for dd in pend:
                dd.wait()
            if k + 1 < DCH:
                rd = read_chunk(k + 1, 1 - b)
            pend = [pltpu.async_copy(rbuf.at[b],
                                     out.at[c, pl.ds(tile_r0 + k * CH, CH)],
                                     wsem)]
        for dd in pend:
            dd.wait()

    return pl.kernel(body, out_type=out_type, mesh=_sc_mesh(),
                     scratch_types=scratch, compiler_params=_SC_PARAMS)


_BN = 2048


def _prep_body(dref, xref, nref, n2ref, xsref, gref):
    dsum = dref[0, :, 0:1] + dref[1, :, 0:1]
    nrm = lax.rsqrt(jnp.maximum(dsum, 1.0))
    nref[...] = nrm
    n2ref[...] = nrm * nrm
    xb = xref[...]
    xsref[0] = xb[:, :16]
    xsref[1] = xb[:, 16:]
    gb = xb * nrm
    gref[0] = gb[:, :16]
    gref[1] = gb[:, 16:]


def _prep_call(deg_part, x_pad):
    grid = (NR // _BN,)
    return pl.pallas_call(
        _prep_body,
        grid=grid,
        in_specs=[
            pl.BlockSpec((2, _BN, DG), lambda i: (0, i, 0)),
            pl.BlockSpec((_BN, 32), lambda i: (i, 0)),
        ],
        out_specs=[
            pl.BlockSpec((_BN, 1), lambda i: (i, 0)),
            pl.BlockSpec((_BN, 1), lambda i: (i, 0)),
            pl.BlockSpec((2, _BN, DG), lambda i: (0, i, 0)),
            pl.BlockSpec((2, _BN, DG), lambda i: (0, i, 0)),
        ],
        out_shape=[
            jax.ShapeDtypeStruct((NR, 1), F32),
            jax.ShapeDtypeStruct((NR, 1), F32),
            jax.ShapeDtypeStruct((2, NR, DG), F32),
            jax.ShapeDtypeStruct((2, NR, DG), F32),
        ],
    )(deg_part, x_pad)


def _make_mm(ng, last):
    def body(href, s1ref, s2ref, wref, bref, nref, *orefs):
        n = nref[...]
        a = bref[...] + jnp.zeros((_BN, 64), F32)
        for g in range(ng):
            a = a + jnp.dot(href[g], wref[g], preferred_element_type=F32)
        for g in range(ng):
            a = a + jnp.dot(s1ref[g] * n, wref[ng + g],
                            preferred_element_type=F32)
        for g in range(ng):
            a = a + jnp.dot(s2ref[g] * n, wref[2 * ng + g],
                            preferred_element_type=F32)
        h = jnp.maximum(a, 0.0)
        for g in range(4):
            orefs[0][g] = h[:, g * DG:(g + 1) * DG]
        if not last:
            hn = h * n
            for g in range(4):
                orefs[1][g] = hn[:, g * DG:(g + 1) * DG]

    grid = (NR // _BN,)
    out_specs = [pl.BlockSpec((4, _BN, DG), lambda i: (0, i, 0))]
    out_shape = [jax.ShapeDtypeStruct((4, NR, DG), F32)]
    if not last:
        out_specs.append(pl.BlockSpec((4, _BN, DG), lambda i: (0, i, 0)))
        out_shape.append(jax.ShapeDtypeStruct((4, NR, DG), F32))

    def call(h_split, s1, s2, wb, bias, norm):
        return pl.pallas_call(
            body,
            grid=grid,
            in_specs=[
                pl.BlockSpec((ng, _BN, DG), lambda i: (0, i, 0)),
                pl.BlockSpec((ng, _BN, DG), lambda i: (0, i, 0)),
                pl.BlockSpec((ng, _BN, DG), lambda i: (0, i, 0)),
                pl.BlockSpec((3 * ng, DG, 64), lambda i: (0, 0, 0)),
                pl.BlockSpec((1, 64), lambda i: (0, 0)),
                pl.BlockSpec((_BN, 1), lambda i: (i, 0)),
            ],
            out_specs=out_specs,
            out_shape=out_shape,
        )(h_split, s1, s2, wb, bias, norm)
    return call


_PBN = 2000


def _pool_body(href, gref, wgref, bgref, oref, gmax_s, den_s, acc_s):
    p = pl.program_id(0)
    i = pl.program_id(1)
    nblk = pl.num_programs(1)
    gate = bgref[...] + jnp.zeros((_PBN, 1), F32)
    for g in range(4):
        gate = gate + jnp.dot(href[g], wgref[g], preferred_element_type=F32)
    ids = gref[...]
    onehot = ids == lax.broadcasted_iota(jnp.int32, (1, GG), 1)

    @pl.when(jnp.logical_and(p == 0, i == 0))
    def _():
        gmax_s[...] = jnp.full((1, GG), -jnp.inf, F32)

    @pl.when(p == 0)
    def _():
        m = jnp.max(jnp.where(onehot, gate, -jnp.inf), axis=0, keepdims=True)
        gmax_s[...] = jnp.maximum(gmax_s[...], m)

    @pl.when(p == 1)
    def _():
        @pl.when(i == 0)
        def _():
            den_s[...] = jnp.zeros((GG, 1), F32)
            acc_s[...] = jnp.zeros((GG, 64), F32)

        gm = gmax_s[...]
        gm = jnp.where(jnp.isfinite(gm), gm, 0.0)
        oh = onehot.astype(F32)
        gnode = lax.dot_general(oh, gm, (((1,), (1,)), ((), ())),
                                preferred_element_type=F32)
        e = jnp.exp(gate - gnode)
        den_s[...] = den_s[...] + lax.dot_general(
            oh, e, (((0,), (0,)), ((), ())), preferred_element_type=F32)
        h = jnp.concatenate([href[0], href[1], href[2], href[3]], axis=1)
        acc_s[...] = acc_s[...] + lax.dot_general(
            oh, e * h, (((0,), (0,)), ((), ())), preferred_element_type=F32)

        @pl.when(i == nblk - 1)
        def _():
            oref[...] = acc_s[...] / jnp.maximum(den_s[...], 1e-12)


def _pool_call(h3s, gid, wg4, bg2):
    grid = (2, NN // _PBN)
    return pl.pallas_call(
        _pool_body,
        grid=grid,
        in_specs=[
            pl.BlockSpec((4, _PBN, DG), lambda p, i: (0, i, 0)),
            pl.BlockSpec((_PBN, 1), lambda p, i: (i, 0)),
            pl.BlockSpec((4, DG, 1), lambda p, i: (0, 0, 0)),
            pl.BlockSpec((1, 1), lambda p, i: (0, 0)),
        ],
        out_specs=pl.BlockSpec((GG, 64), lambda p, i: (0, 0)),
        out_shape=jax.ShapeDtypeStruct((GG, 64), F32),
        scratch_shapes=[
            pltpu.VMEM((1, GG), F32),
            pltpu.VMEM((GG, 1), F32),
            pltpu.VMEM((GG, 64), F32),
        ],
    )(h3s, gid, wg4, bg2)


@jax.jit
def kernel(x, edge_index, graph_ids, W0, b0, W1, b1, W2, b2, W3, b3, Wg, bg):
    src = edge_index[0]
    dst = edge_index[1]
    epad = EP - EE
    src_pad = jnp.concatenate([src, jnp.zeros((epad,), jnp.int32)])
    dst_pad = jnp.concatenate([dst, jnp.full((epad,), NN, jnp.int32)])
    src2 = src_pad.reshape(TCH, CH)
    dst2 = dst_pad.reshape(TCH, CH)
    x_pad = jnp.pad(x, ((0, NR - NN), (0, 1)))

    ridx = jnp.concatenate([jnp.arange(0, 31), jnp.arange(32, 63),
                            jnp.arange(64, 95)])
    wb0 = jnp.zeros((96, 64), F32).at[ridx].set(W0).reshape(6, DG, 64)
    wbs = [wb0, W1.reshape(12, DG, 64), W2.reshape(12, DG, 64),
           W3.reshape(12, DG, 64)]
    biases = [b0.reshape(1, 64), b1.reshape(1, 64), b2.reshape(1, 64),
              b3.reshape(1, 64)]

    deg_part = _make_deg()(dst2)
    norm, norm2, h_split, g = _prep_call(deg_part, x_pad)
    norm2f = norm2.reshape(NR)

    prop2_2 = _make_prop(2, True)
    prop1_2 = _make_prop(2, False)
    prop2_4 = _make_prop(4, True)
    prop1_4 = _make_prop(4, False)
    mm0 = _make_mm(2, False)
    mm = _make_mm(4, False)
    mm_last = _make_mm(4, True)

    ng = 2
    for layer in range(4):
        p2 = prop2_2 if ng == 2 else prop2_4
        p1 = prop1_2 if ng == 2 else prop1_4
        s1, g2 = p2(g, src2, dst2, norm2f)
        (s2,) = p1(g2, src2, dst2)
        if layer == 0:
            h_split, g = mm0(h_split, s1, s2, wbs[0], biases[0], norm)
        elif layer < 3:
            h_split, g = mm(h_split, s1, s2, wbs[layer], biases[layer], norm)
        else:
            (h_split,) = mm_last(h_split, s1, s2, wbs[3], biases[3], norm)
        ng = 4

    out = _pool_call(h_split, graph_ids.reshape(NN, 1),
                     Wg.reshape(4, DG, 1), bg.reshape(1, 1))
    return out.reshape(10, 50, 64)
